# Initial kernel scaffold; baseline (speedup 1.0000x reference)
import jax
import jax.numpy as jnp
from jax.experimental import pallas as pl


def kernel(x, adj_t, W1, b1, W2, b2, W3, b3):
    raise NotImplementedError("write your pallas kernel here")


# speedup vs baseline: 13.3038x; 13.3038x over previous
import functools

import jax
import jax.numpy as jnp
from jax import lax
from jax.experimental import pallas as pl
from jax.experimental.pallas import tpu as pltpu
from jax.experimental.pallas import tpu_sc as plsc

N = 10000
E = 320000
D_IN = 128
D_H = 128
D_OUT = 64

NC = 2
NS = 16
NW = NC * NS
CHUNK = 128
EPW = 10240
E_PAD = NW * EPW
N_PAD = 10240
ROWS_PER_TILE = N_PAD // NS
NCHUNK = EPW // CHUNK
DEG_W = 128


def _make_scatter(D):
    mesh = plsc.VectorSubcoreMesh(core_axis_name="c", subcore_axis_name="s")
    params = None if D % 128 == 0 else pltpu.CompilerParams(
        use_tc_tiling_on_sc=False)

    @functools.partial(
        pl.kernel,
        out_type=jax.ShapeDtypeStruct((NC, N_PAD, D), jnp.float32),
        mesh=mesh,
        compiler_params=params,
        scratch_types=[
            pltpu.VMEM((CHUNK,), jnp.int32),
            pltpu.VMEM((CHUNK,), jnp.int32),
            pltpu.VMEM((CHUNK, D), jnp.float32),
            pltpu.VMEM_SHARED((N_PAD, D), jnp.float32),
            pltpu.SemaphoreType.DMA,
        ],
    )
    def scatter_kernel(y_hbm, src_hbm, dst_hbm, zero_hbm, out_hbm,
                       src_v, dst_v, rows_v, z_sh, sem):
        c = lax.axis_index("c")
        s = lax.axis_index("s")
        wid = s * NC + c
        row0 = s * ROWS_PER_TILE
        pltpu.sync_copy(zero_hbm.at[pl.ds(row0, ROWS_PER_TILE)],
                        z_sh.at[pl.ds(row0, ROWS_PER_TILE)])
        plsc.subcore_barrier()
        base = wid * EPW

        def body(i, _):
            off = base + i * CHUNK
            pltpu.sync_copy(src_hbm.at[pl.ds(off, CHUNK)], src_v)
            pltpu.sync_copy(dst_hbm.at[pl.ds(off, CHUNK)], dst_v)
            pltpu.async_copy(y_hbm.at[src_v], rows_v, sem).wait()
            pltpu.sync_copy(rows_v, z_sh.at[dst_v], add=True)
            return ()

        lax.fori_loop(0, NCHUNK, body, ())
        plsc.subcore_barrier()
        pltpu.sync_copy(z_sh.at[pl.ds(row0, ROWS_PER_TILE)],
                        out_hbm.at[c, pl.ds(row0, ROWS_PER_TILE)])

    return scatter_kernel


def _make_degree():
    mesh = plsc.VectorSubcoreMesh(core_axis_name="c", subcore_axis_name="s")

    @functools.partial(
        pl.kernel,
        out_type=jax.ShapeDtypeStruct((NC, N_PAD, DEG_W), jnp.float32),
        mesh=mesh,
        scratch_types=[
            pltpu.VMEM((CHUNK,), jnp.int32),
            pltpu.VMEM((CHUNK, DEG_W), jnp.float32),
            pltpu.VMEM_SHARED((N_PAD, DEG_W), jnp.float32),
        ],
    )
    def degree_kernel(dst_hbm, ones_hbm, zero_hbm, out_hbm,
                      dst_v, ones_v, deg_sh):
        c = lax.axis_index("c")
        s = lax.axis_index("s")
        wid = s * NC + c
        row0 = s * ROWS_PER_TILE
        pltpu.sync_copy(ones_hbm, ones_v)
        pltpu.sync_copy(zero_hbm.at[pl.ds(row0, ROWS_PER_TILE)],
                        deg_sh.at[pl.ds(row0, ROWS_PER_TILE)])
        plsc.subcore_barrier()
        base = wid * EPW

        def body(i, _):
            off = base + i * CHUNK
            pltpu.sync_copy(dst_hbm.at[pl.ds(off, CHUNK)], dst_v)
            pltpu.sync_copy(ones_v, deg_sh.at[dst_v], add=True)
            return ()

        lax.fori_loop(0, NCHUNK, body, ())
        plsc.subcore_barrier()
        pltpu.sync_copy(deg_sh.at[pl.ds(row0, ROWS_PER_TILE)],
                        out_hbm.at[c, pl.ds(row0, ROWS_PER_TILE)])

    return degree_kernel


_BR = 1000


def _tc_first(x, W1, deg0, deg1):
    def body(x_ref, w_ref, d0_ref, d1_ref, y_ref, dinv_ref):
        deg = d0_ref[...] + d1_ref[...] + 1.0
        dinv = lax.rsqrt(deg)
        xw = jnp.dot(x_ref[...], w_ref[...],
                     preferred_element_type=jnp.float32,
                     precision=lax.Precision.HIGHEST)
        y_ref[...] = dinv * xw
        dinv_ref[...] = dinv

    return pl.pallas_call(
        body,
        grid=(N // _BR,),
        in_specs=[
            pl.BlockSpec((_BR, D_IN), lambda i: (i, 0)),
            pl.BlockSpec((D_IN, D_H), lambda i: (0, 0)),
            pl.BlockSpec((_BR, 1), lambda i: (i, 0)),
            pl.BlockSpec((_BR, 1), lambda i: (i, 0)),
        ],
        out_specs=[
            pl.BlockSpec((_BR, D_H), lambda i: (i, 0)),
            pl.BlockSpec((_BR, 1), lambda i: (i, 0)),
        ],
        out_shape=[
            jax.ShapeDtypeStruct((N, D_H), jnp.float32),
            jax.ShapeDtypeStruct((N, 1), jnp.float32),
        ],
    )(x, W1, deg0, deg1)


def _tc_mid(z0, z1, y, dinv, b, W, d_out):
    d_in = y.shape[1]

    def body(z0_ref, z1_ref, y_ref, dinv_ref, b_ref, w_ref, out_ref):
        dv = dinv_ref[...]
        h = jnp.maximum(dv * (z0_ref[...] + z1_ref[...] + y_ref[...]) + b_ref[...],
                        0.0)
        out_ref[...] = dv * jnp.dot(h, w_ref[...],
                                    preferred_element_type=jnp.float32,
                                    precision=lax.Precision.HIGHEST)

    return pl.pallas_call(
        body,
        grid=(N // _BR,),
        in_specs=[
            pl.BlockSpec((_BR, d_in), lambda i: (i, 0)),
            pl.BlockSpec((_BR, d_in), lambda i: (i, 0)),
            pl.BlockSpec((_BR, d_in), lambda i: (i, 0)),
            pl.BlockSpec((_BR, 1), lambda i: (i, 0)),
            pl.BlockSpec((1, d_in), lambda i: (0, 0)),
            pl.BlockSpec((d_in, d_out), lambda i: (0, 0)),
        ],
        out_specs=pl.BlockSpec((_BR, d_out), lambda i: (i, 0)),
        out_shape=jax.ShapeDtypeStruct((N, d_out), jnp.float32),
    )(z0, z1, y, dinv, b, W)


def _tc_last(z0, z1, y, dinv, b):
    def body(z0_ref, z1_ref, y_ref, dinv_ref, b_ref, out_ref):
        o = dinv_ref[...] * (z0_ref[...] + z1_ref[...] + y_ref[...]) + b_ref[...]
        m = jnp.max(o, axis=-1, keepdims=True)
        t = o - m
        out_ref[...] = t - jnp.log(jnp.sum(jnp.exp(t), axis=-1, keepdims=True))

    return pl.pallas_call(
        body,
        grid=(N // _BR,),
        in_specs=[
            pl.BlockSpec((_BR, D_OUT), lambda i: (i, 0)),
            pl.BlockSpec((_BR, D_OUT), lambda i: (i, 0)),
            pl.BlockSpec((_BR, D_OUT), lambda i: (i, 0)),
            pl.BlockSpec((_BR, 1), lambda i: (i, 0)),
            pl.BlockSpec((1, D_OUT), lambda i: (0, 0)),
        ],
        out_specs=pl.BlockSpec((_BR, D_OUT), lambda i: (i, 0)),
        out_shape=jax.ShapeDtypeStruct((N, D_OUT), jnp.float32),
    )(z0, z1, y, dinv, b)


def kernel(x, adj_t, W1, b1, W2, b2, W3, b3):
    pad = E_PAD - E
    pad_src = (jnp.arange(pad, dtype=jnp.int32) * 131) % N
    pad_dst = N + (jnp.arange(pad, dtype=jnp.int32) % (N_PAD - N))
    src = jnp.concatenate([adj_t[0], pad_src])
    dst = jnp.concatenate([adj_t[1], pad_dst])

    zeros128 = jnp.zeros((N_PAD, D_H), jnp.float32)
    zeros64 = jnp.zeros((N_PAD, D_OUT), jnp.float32)
    zeros_deg = jnp.zeros((N_PAD, DEG_W), jnp.float32)
    ones_deg = jnp.ones((CHUNK, DEG_W), jnp.float32)

    degree_k = _make_degree()
    scatter128 = _make_scatter(D_H)
    scatter64 = _make_scatter(D_OUT)

    degt = degree_k(dst, ones_deg, zeros_deg)
    deg0 = degt[0, :N, 0:1]
    deg1 = degt[1, :N, 0:1]

    y1, dinv = _tc_first(x, W1, deg0, deg1)
    z1 = scatter128(y1, src, dst, zeros128)
    y2 = _tc_mid(z1[0, :N], z1[1, :N], y1, dinv, b1.reshape(1, D_H), W2, D_H)
    z2 = scatter128(y2, src, dst, zeros128)
    y3 = _tc_mid(z2[0, :N], z2[1, :N], y2, dinv, b2.reshape(1, D_H), W3, D_OUT)
    z3 = scatter64(y3, src, dst, zeros64)

    return _tc_last(z3[0, :N], z3[1, :N], y3, dinv, b3.reshape(1, D_OUT))


# speedup vs baseline: 21.4165x; 1.6098x over previous
import functools

import jax
import jax.numpy as jnp
from jax import lax
from jax.experimental import pallas as pl
from jax.experimental.pallas import tpu as pltpu
from jax.experimental.pallas import tpu_sc as plsc

N = 10000
E = 320000
D_IN = 128
D_H = 128
D_OUT = 64

NC = 2
NS = 16
NW = NC * NS
CHUNK = 128
EPW = 10240
E_PAD = NW * EPW
N_PAD = 10240
ROWS_PER_TILE = N_PAD // NS
NCHUNK = EPW // CHUNK
DEG_W = 128


def _make_scatter(D):
    mesh = plsc.VectorSubcoreMesh(core_axis_name="c", subcore_axis_name="s")
    params = None if D % 128 == 0 else pltpu.CompilerParams(
        use_tc_tiling_on_sc=False)

    @functools.partial(
        pl.kernel,
        out_type=jax.ShapeDtypeStruct((NC, N_PAD, D), jnp.float32),
        mesh=mesh,
        compiler_params=params,
        scratch_types=[
            [pltpu.VMEM((CHUNK,), jnp.int32)] * 2,
            [pltpu.VMEM((CHUNK,), jnp.int32)] * 2,
            [pltpu.VMEM((CHUNK, D), jnp.float32)] * 2,
            pltpu.VMEM_SHARED((N_PAD, D), jnp.float32),
            [pltpu.SemaphoreType.DMA] * 2,
            [pltpu.SemaphoreType.DMA] * 2,
        ],
    )
    def scatter_kernel(y_hbm, src_hbm, dst_hbm, zero_hbm, out_hbm,
                       src_v, dst_v, rows_v, z_sh, isem, gsem):
        c = lax.axis_index("c")
        s = lax.axis_index("s")
        wid = s * NC + c
        row0 = s * ROWS_PER_TILE
        base = wid * EPW

        def idx_start(ch, b):
            off = base + ch * CHUNK
            pltpu.make_async_copy(src_hbm.at[pl.ds(off, CHUNK)],
                                  src_v[b], isem[b]).start()
            pltpu.make_async_copy(dst_hbm.at[pl.ds(off, CHUNK)],
                                  dst_v[b], isem[b]).start()

        def idx_wait(b):
            pltpu.make_async_copy(src_hbm.at[pl.ds(0, CHUNK)],
                                  src_v[b], isem[b]).wait()
            pltpu.make_async_copy(dst_hbm.at[pl.ds(0, CHUNK)],
                                  dst_v[b], isem[b]).wait()

        def gather_start(b):
            pltpu.make_async_copy(y_hbm.at[src_v[b]], rows_v[b],
                                  gsem[b]).start()

        def gather_wait(b):
            pltpu.make_async_copy(y_hbm.at[src_v[b]], rows_v[b],
                                  gsem[b]).wait()

        idx_start(0, 0)
        idx_start(1, 1)
        pltpu.sync_copy(zero_hbm.at[pl.ds(row0, ROWS_PER_TILE)],
                        z_sh.at[pl.ds(row0, ROWS_PER_TILE)])
        idx_wait(0)
        gather_start(0)
        plsc.subcore_barrier()

        def process(ch, b, pf_gather, pf_idx):
            if pf_gather:
                idx_wait(1 - b)
                gather_start(1 - b)
            gather_wait(b)
            pltpu.sync_copy(rows_v[b], z_sh.at[dst_v[b]], add=True)
            if pf_idx:
                idx_start(ch + 2, b)

        def body(g, _):
            process(g, 0, True, True)
            process(g + 1, 1, True, True)
            return ()

        lax.fori_loop(0, (NCHUNK - 4) // 2, lambda i, _: body(i * 2, _), ())
        process(NCHUNK - 4, 0, True, True)
        process(NCHUNK - 3, 1, True, True)
        process(NCHUNK - 2, 0, True, False)
        process(NCHUNK - 1, 1, False, False)
        plsc.subcore_barrier()
        pltpu.sync_copy(z_sh.at[pl.ds(row0, ROWS_PER_TILE)],
                        out_hbm.at[c, pl.ds(row0, ROWS_PER_TILE)])

    return scatter_kernel


def _make_degree():
    mesh = plsc.VectorSubcoreMesh(core_axis_name="c", subcore_axis_name="s")

    @functools.partial(
        pl.kernel,
        out_type=jax.ShapeDtypeStruct((NC, N_PAD, DEG_W), jnp.float32),
        mesh=mesh,
        scratch_types=[
            pltpu.VMEM((CHUNK,), jnp.int32),
            pltpu.VMEM((CHUNK, DEG_W), jnp.float32),
            pltpu.VMEM_SHARED((N_PAD, DEG_W), jnp.float32),
        ],
    )
    def degree_kernel(dst_hbm, ones_hbm, zero_hbm, out_hbm,
                      dst_v, ones_v, deg_sh):
        c = lax.axis_index("c")
        s = lax.axis_index("s")
        wid = s * NC + c
        row0 = s * ROWS_PER_TILE
        pltpu.sync_copy(ones_hbm, ones_v)
        pltpu.sync_copy(zero_hbm.at[pl.ds(row0, ROWS_PER_TILE)],
                        deg_sh.at[pl.ds(row0, ROWS_PER_TILE)])
        plsc.subcore_barrier()
        base = wid * EPW

        def body(i, _):
            off = base + i * CHUNK
            pltpu.sync_copy(dst_hbm.at[pl.ds(off, CHUNK)], dst_v)
            pltpu.sync_copy(ones_v, deg_sh.at[dst_v], add=True)
            return ()

        lax.fori_loop(0, NCHUNK, body, ())
        plsc.subcore_barrier()
        pltpu.sync_copy(deg_sh.at[pl.ds(row0, ROWS_PER_TILE)],
                        out_hbm.at[c, pl.ds(row0, ROWS_PER_TILE)])

    return degree_kernel


_BR = 1000


def _tc_first(x, W1, deg0, deg1):
    def body(x_ref, w_ref, d0_ref, d1_ref, y_ref, dinv_ref):
        deg = d0_ref[...] + d1_ref[...] + 1.0
        dinv = lax.rsqrt(deg)
        xw = jnp.dot(x_ref[...], w_ref[...],
                     preferred_element_type=jnp.float32,
                     precision=lax.Precision.HIGHEST)
        y_ref[...] = dinv * xw
        dinv_ref[...] = dinv

    return pl.pallas_call(
        body,
        grid=(N // _BR,),
        in_specs=[
            pl.BlockSpec((_BR, D_IN), lambda i: (i, 0)),
            pl.BlockSpec((D_IN, D_H), lambda i: (0, 0)),
            pl.BlockSpec((_BR, 1), lambda i: (i, 0)),
            pl.BlockSpec((_BR, 1), lambda i: (i, 0)),
        ],
        out_specs=[
            pl.BlockSpec((_BR, D_H), lambda i: (i, 0)),
            pl.BlockSpec((_BR, 1), lambda i: (i, 0)),
        ],
        out_shape=[
            jax.ShapeDtypeStruct((N, D_H), jnp.float32),
            jax.ShapeDtypeStruct((N, 1), jnp.float32),
        ],
    )(x, W1, deg0, deg1)


def _tc_mid(z0, z1, y, dinv, b, W, d_out):
    d_in = y.shape[1]

    def body(z0_ref, z1_ref, y_ref, dinv_ref, b_ref, w_ref, out_ref):
        dv = dinv_ref[...]
        h = jnp.maximum(dv * (z0_ref[...] + z1_ref[...] + y_ref[...]) + b_ref[...],
                        0.0)
        out_ref[...] = dv * jnp.dot(h, w_ref[...],
                                    preferred_element_type=jnp.float32,
                                    precision=lax.Precision.HIGHEST)

    return pl.pallas_call(
        body,
        grid=(N // _BR,),
        in_specs=[
            pl.BlockSpec((_BR, d_in), lambda i: (i, 0)),
            pl.BlockSpec((_BR, d_in), lambda i: (i, 0)),
            pl.BlockSpec((_BR, d_in), lambda i: (i, 0)),
            pl.BlockSpec((_BR, 1), lambda i: (i, 0)),
            pl.BlockSpec((1, d_in), lambda i: (0, 0)),
            pl.BlockSpec((d_in, d_out), lambda i: (0, 0)),
        ],
        out_specs=pl.BlockSpec((_BR, d_out), lambda i: (i, 0)),
        out_shape=jax.ShapeDtypeStruct((N, d_out), jnp.float32),
    )(z0, z1, y, dinv, b, W)


def _tc_last(z0, z1, y, dinv, b):
    def body(z0_ref, z1_ref, y_ref, dinv_ref, b_ref, out_ref):
        o = dinv_ref[...] * (z0_ref[...] + z1_ref[...] + y_ref[...]) + b_ref[...]
        m = jnp.max(o, axis=-1, keepdims=True)
        t = o - m
        out_ref[...] = t - jnp.log(jnp.sum(jnp.exp(t), axis=-1, keepdims=True))

    return pl.pallas_call(
        body,
        grid=(N // _BR,),
        in_specs=[
            pl.BlockSpec((_BR, D_OUT), lambda i: (i, 0)),
            pl.BlockSpec((_BR, D_OUT), lambda i: (i, 0)),
            pl.BlockSpec((_BR, D_OUT), lambda i: (i, 0)),
            pl.BlockSpec((_BR, 1), lambda i: (i, 0)),
            pl.BlockSpec((1, D_OUT), lambda i: (0, 0)),
        ],
        out_specs=pl.BlockSpec((_BR, D_OUT), lambda i: (i, 0)),
        out_shape=jax.ShapeDtypeStruct((N, D_OUT), jnp.float32),
    )(z0, z1, y, dinv, b)


def kernel(x, adj_t, W1, b1, W2, b2, W3, b3):
    pad = E_PAD - E
    pad_src = (jnp.arange(pad, dtype=jnp.int32) * 131) % N
    pad_dst = N + (jnp.arange(pad, dtype=jnp.int32) % (N_PAD - N))
    src = jnp.concatenate([adj_t[0], pad_src])
    dst = jnp.concatenate([adj_t[1], pad_dst])

    zeros128 = jnp.zeros((N_PAD, D_H), jnp.float32)
    zeros64 = jnp.zeros((N_PAD, D_OUT), jnp.float32)
    zeros_deg = jnp.zeros((N_PAD, DEG_W), jnp.float32)
    ones_deg = jnp.ones((CHUNK, DEG_W), jnp.float32)

    degree_k = _make_degree()
    scatter128 = _make_scatter(D_H)
    scatter64 = _make_scatter(D_OUT)

    degt = degree_k(dst, ones_deg, zeros_deg)
    deg0 = degt[0, :N, 0:1]
    deg1 = degt[1, :N, 0:1]

    y1, dinv = _tc_first(x, W1, deg0, deg1)
    z1 = scatter128(y1, src, dst, zeros128)
    y2 = _tc_mid(z1[0, :N], z1[1, :N], y1, dinv, b1.reshape(1, D_H), W2, D_H)
    z2 = scatter128(y2, src, dst, zeros128)
    y3 = _tc_mid(z2[0, :N], z2[1, :N], y2, dinv, b2.reshape(1, D_H), W3, D_OUT)
    z3 = scatter64(y3, src, dst, zeros64)

    return _tc_last(z3[0, :N], z3[1, :N], y3, dinv, b3.reshape(1, D_OUT))


# speedup vs baseline: 26.4434x; 1.2347x over previous
import functools

import jax
import jax.numpy as jnp
from jax import lax
from jax.experimental import pallas as pl
from jax.experimental.pallas import tpu as pltpu
from jax.experimental.pallas import tpu_sc as plsc

N = 10000
E = 320000
D_IN = 128
D_H = 128
D_OUT = 64

NC = 2
NS = 16
NW = NC * NS
CHUNK = 128
EPW = 10240
E_PAD = NW * EPW
N_PAD = 10240
ROWS_PER_TILE = N_PAD // NS
NCHUNK = EPW // CHUNK
DEG_W = 128


def _make_scatter(D):
    mesh = plsc.VectorSubcoreMesh(core_axis_name="c", subcore_axis_name="s")
    params = None if D % 128 == 0 else pltpu.CompilerParams(
        use_tc_tiling_on_sc=False)

    @functools.partial(
        pl.kernel,
        out_type=jax.ShapeDtypeStruct((NC, N_PAD, D), jnp.float32),
        mesh=mesh,
        compiler_params=params,
        scratch_types=[
            [pltpu.VMEM((CHUNK,), jnp.int32)] * 2,
            [pltpu.VMEM((CHUNK,), jnp.int32)] * 2,
            [pltpu.VMEM((CHUNK, D), jnp.float32)] * 2,
            pltpu.VMEM_SHARED((N_PAD, D), jnp.float32),
            [pltpu.SemaphoreType.DMA] * 2,
            [pltpu.SemaphoreType.DMA] * 2,
        ],
    )
    def scatter_kernel(y_hbm, src_hbm, dst_hbm, zero_hbm, out_hbm,
                       src_v, dst_v, rows_v, z_sh, isem, gsem):
        c = lax.axis_index("c")
        s = lax.axis_index("s")
        wid = s * NC + c
        row0 = s * ROWS_PER_TILE
        base = wid * EPW

        def idx_start(ch, b):
            off = base + ch * CHUNK
            pltpu.make_async_copy(src_hbm.at[pl.ds(off, CHUNK)],
                                  src_v[b], isem[b]).start()
            pltpu.make_async_copy(dst_hbm.at[pl.ds(off, CHUNK)],
                                  dst_v[b], isem[b]).start()

        def idx_wait(b):
            pltpu.make_async_copy(src_hbm.at[pl.ds(0, CHUNK)],
                                  src_v[b], isem[b]).wait()
            pltpu.make_async_copy(dst_hbm.at[pl.ds(0, CHUNK)],
                                  dst_v[b], isem[b]).wait()

        def gather_start(b):
            pltpu.make_async_copy(y_hbm.at[src_v[b]], rows_v[b],
                                  gsem[b]).start()

        def gather_wait(b):
            pltpu.make_async_copy(y_hbm.at[src_v[b]], rows_v[b],
                                  gsem[b]).wait()

        idx_start(0, 0)
        idx_start(1, 1)
        pltpu.sync_copy(zero_hbm.at[pl.ds(row0, ROWS_PER_TILE)],
                        z_sh.at[pl.ds(row0, ROWS_PER_TILE)])
        idx_wait(0)
        gather_start(0)
        plsc.subcore_barrier()

        def process(ch, b, pf_gather, pf_idx):
            if pf_gather:
                idx_wait(1 - b)
                gather_start(1 - b)
            gather_wait(b)
            pltpu.sync_copy(rows_v[b], z_sh.at[dst_v[b]], add=True)
            if pf_idx:
                idx_start(ch + 2, b)

        def body(g, _):
            process(g, 0, True, True)
            process(g + 1, 1, True, True)
            return ()

        lax.fori_loop(0, (NCHUNK - 4) // 2, lambda i, _: body(i * 2, _), ())
        process(NCHUNK - 4, 0, True, True)
        process(NCHUNK - 3, 1, True, True)
        process(NCHUNK - 2, 0, True, False)
        process(NCHUNK - 1, 1, False, False)
        plsc.subcore_barrier()
        pltpu.sync_copy(z_sh.at[pl.ds(row0, ROWS_PER_TILE)],
                        out_hbm.at[c, pl.ds(row0, ROWS_PER_TILE)])

    return scatter_kernel


def _make_degree():
    mesh = plsc.VectorSubcoreMesh(core_axis_name="c", subcore_axis_name="s")

    @functools.partial(
        pl.kernel,
        out_type=jax.ShapeDtypeStruct((NW, N_PAD), jnp.float32),
        mesh=mesh,
        compiler_params=pltpu.CompilerParams(needs_layout_passes=False),
        scratch_types=[
            pltpu.VMEM((N_PAD,), jnp.float32),
            pltpu.VMEM((EPW,), jnp.int32),
        ],
    )
    def degree_kernel(dst_hbm, out_hbm, hist_v, dst_v):
        c = lax.axis_index("c")
        s = lax.axis_index("s")
        wid = s * NC + c
        pltpu.sync_copy(dst_hbm.at[pl.ds(wid * EPW, EPW)], dst_v)

        def zbody(i, _):
            hist_v[pl.ds(i * 16, 16)] = jnp.zeros((16,), jnp.float32)
            return ()

        lax.fori_loop(0, N_PAD // 16, zbody, ())
        ones16 = jnp.ones((16,), jnp.float32)

        def body(j, _):
            idx = dst_v[pl.ds(j * 16, 16)]
            plsc.addupdate_scatter(hist_v, [idx], ones16)
            return ()

        lax.fori_loop(0, EPW // 16, body, ())
        pltpu.sync_copy(hist_v, out_hbm.at[wid])

    return degree_kernel


_BR = 1024


def _tc_first(x, W1, deg_parts):
    def body(x_ref, w_ref, dp_ref, y_ref, dinv_ref):
        deg = jnp.sum(dp_ref[...], axis=0) + 1.0
        dinv = lax.rsqrt(deg)[:, None]
        xw = jnp.dot(x_ref[...], w_ref[...],
                     preferred_element_type=jnp.float32,
                     precision=lax.Precision.HIGHEST)
        y_ref[...] = dinv * xw
        dinv_ref[...] = dinv

    return pl.pallas_call(
        body,
        grid=(N_PAD // _BR,),
        in_specs=[
            pl.BlockSpec((_BR, D_IN), lambda i: (i, 0)),
            pl.BlockSpec((D_IN, D_H), lambda i: (0, 0)),
            pl.BlockSpec((NW, _BR), lambda i: (0, i)),
        ],
        out_specs=[
            pl.BlockSpec((_BR, D_H), lambda i: (i, 0)),
            pl.BlockSpec((_BR, 1), lambda i: (i, 0)),
        ],
        out_shape=[
            jax.ShapeDtypeStruct((N_PAD, D_H), jnp.float32),
            jax.ShapeDtypeStruct((N_PAD, 1), jnp.float32),
        ],
    )(x, W1, deg_parts)


def _tc_mid(z0, z1, y, dinv, b, W, d_out):
    d_in = y.shape[1]

    def body(z0_ref, z1_ref, y_ref, dinv_ref, b_ref, w_ref, out_ref):
        dv = dinv_ref[...]
        h = jnp.maximum(dv * (z0_ref[...] + z1_ref[...] + y_ref[...]) + b_ref[...],
                        0.0)
        out_ref[...] = dv * jnp.dot(h, w_ref[...],
                                    preferred_element_type=jnp.float32,
                                    precision=lax.Precision.HIGHEST)

    return pl.pallas_call(
        body,
        grid=(N_PAD // _BR,),
        in_specs=[
            pl.BlockSpec((_BR, d_in), lambda i: (i, 0)),
            pl.BlockSpec((_BR, d_in), lambda i: (i, 0)),
            pl.BlockSpec((_BR, d_in), lambda i: (i, 0)),
            pl.BlockSpec((_BR, 1), lambda i: (i, 0)),
            pl.BlockSpec((1, d_in), lambda i: (0, 0)),
            pl.BlockSpec((d_in, d_out), lambda i: (0, 0)),
        ],
        out_specs=pl.BlockSpec((_BR, d_out), lambda i: (i, 0)),
        out_shape=jax.ShapeDtypeStruct((N_PAD, d_out), jnp.float32),
    )(z0, z1, y, dinv, b, W)


def _tc_last(z0, z1, y, dinv, b):
    def body(z0_ref, z1_ref, y_ref, dinv_ref, b_ref, out_ref):
        o = dinv_ref[...] * (z0_ref[...] + z1_ref[...] + y_ref[...]) + b_ref[...]
        m = jnp.max(o, axis=-1, keepdims=True)
        t = o - m
        out_ref[...] = t - jnp.log(jnp.sum(jnp.exp(t), axis=-1, keepdims=True))

    return pl.pallas_call(
        body,
        grid=(N_PAD // _BR,),
        in_specs=[
            pl.BlockSpec((_BR, D_OUT), lambda i: (i, 0)),
            pl.BlockSpec((_BR, D_OUT), lambda i: (i, 0)),
            pl.BlockSpec((_BR, D_OUT), lambda i: (i, 0)),
            pl.BlockSpec((_BR, 1), lambda i: (i, 0)),
            pl.BlockSpec((1, D_OUT), lambda i: (0, 0)),
        ],
        out_specs=pl.BlockSpec((_BR, D_OUT), lambda i: (i, 0)),
        out_shape=jax.ShapeDtypeStruct((N_PAD, D_OUT), jnp.float32),
    )(z0, z1, y, dinv, b)


def kernel(x, adj_t, W1, b1, W2, b2, W3, b3):
    pad = E_PAD - E
    pad_src = (jnp.arange(pad, dtype=jnp.int32) * 131) % N
    pad_dst = N + (jnp.arange(pad, dtype=jnp.int32) % (N_PAD - N))
    src = jnp.concatenate([adj_t[0], pad_src])
    dst = jnp.concatenate([adj_t[1], pad_dst])

    zeros128 = jnp.zeros((N_PAD, D_H), jnp.float32)
    zeros64 = jnp.zeros((N_PAD, D_OUT), jnp.float32)

    degree_k = _make_degree()
    scatter128 = _make_scatter(D_H)
    scatter64 = _make_scatter(D_OUT)

    deg_parts = degree_k(dst)
    x_pad = jnp.pad(x, ((0, N_PAD - N), (0, 0)))

    y1, dinv = _tc_first(x_pad, W1, deg_parts)
    z1 = scatter128(y1, src, dst, zeros128)
    y2 = _tc_mid(z1[0], z1[1], y1, dinv, b1.reshape(1, D_H), W2, D_H)
    z2 = scatter128(y2, src, dst, zeros128)
    y3 = _tc_mid(z2[0], z2[1], y2, dinv, b2.reshape(1, D_H), W3, D_OUT)
    z3 = scatter64(y3, src, dst, zeros64)

    return _tc_last(z3[0], z3[1], y3, dinv, b3.reshape(1, D_OUT))[:N]


# speedup vs baseline: 32.3385x; 1.2229x over previous
import functools

import jax
import jax.numpy as jnp
from jax import lax
from jax.experimental import pallas as pl
from jax.experimental.pallas import tpu as pltpu
from jax.experimental.pallas import tpu_sc as plsc

N = 10000
E = 320000
D_IN = 128
D_H = 128
D_OUT = 64

NC = 2
NS = 16
NW = NC * NS
CHUNK = 128
EPW = E // NW
NCHUNK = EPW // CHUNK
TAIL = EPW - NCHUNK * CHUNK
N_ACC = 10112
ROWS_PER_TILE = N_ACC // NS


def _make_scatter(D):
    mesh = plsc.VectorSubcoreMesh(core_axis_name="c", subcore_axis_name="s")
    params = None if D % 128 == 0 else pltpu.CompilerParams(
        use_tc_tiling_on_sc=False)

    @functools.partial(
        pl.kernel,
        out_type=jax.ShapeDtypeStruct((NC, N_ACC, D), jnp.float32),
        mesh=mesh,
        compiler_params=params,
        scratch_types=[
            [pltpu.VMEM((CHUNK,), jnp.int32)] * 2,
            pltpu.VMEM((NCHUNK, CHUNK), jnp.int32),
            [pltpu.VMEM((CHUNK, D), jnp.float32)] * 2,
            pltpu.VMEM((TAIL,), jnp.int32),
            pltpu.VMEM((TAIL,), jnp.int32),
            pltpu.VMEM((TAIL, D), jnp.float32),
            pltpu.VMEM_SHARED((N_ACC, D), jnp.float32),
            pltpu.SemaphoreType.DMA,
            [pltpu.SemaphoreType.DMA] * 2,
            [pltpu.SemaphoreType.DMA] * 2,
            [pltpu.SemaphoreType.DMA] * 2,
        ],
    )
    def scatter_kernel(y_hbm, src_hbm, dst_hbm, zero_hbm, out_hbm,
                       src_v, dst_v, rows_v, stail_v, dtail_v, rtail_v,
                       z_sh, psem, isem, gsem, ssem):
        c = lax.axis_index("c")
        s = lax.axis_index("s")
        wid = s * NC + c
        row0 = s * ROWS_PER_TILE
        base = wid * EPW

        def idx_start(ch, b):
            pltpu.make_async_copy(src_hbm.at[pl.ds(base + ch * CHUNK, CHUNK)],
                                  src_v[b], isem[b]).start()

        def idx_wait(b):
            pltpu.make_async_copy(src_hbm.at[pl.ds(0, CHUNK)],
                                  src_v[b], isem[b]).wait()

        def gather_start(b):
            pltpu.make_async_copy(y_hbm.at[src_v[b]], rows_v[b],
                                  gsem[b]).start()

        def gather_wait(b):
            pltpu.make_async_copy(y_hbm.at[src_v[b]], rows_v[b],
                                  gsem[b]).wait()

        def scatter_start(ch, b):
            pltpu.make_async_copy(rows_v[b], z_sh.at[dst_v.at[ch]],
                                  ssem[b]).start(add=True)

        def scatter_wait(ch, b):
            pltpu.make_async_copy(rows_v[b], z_sh.at[dst_v.at[ch]],
                                  ssem[b]).wait()

        for i in range(NCHUNK):
            pltpu.make_async_copy(dst_hbm.at[pl.ds(base + i * CHUNK, CHUNK)],
                                  dst_v.at[i], psem).start()
        toff = base + NCHUNK * CHUNK
        pltpu.make_async_copy(src_hbm.at[pl.ds(toff, TAIL)], stail_v,
                              psem).start()
        pltpu.make_async_copy(dst_hbm.at[pl.ds(toff, TAIL)], dtail_v,
                              psem).start()
        idx_start(0, 0)
        idx_start(1, 1)
        pltpu.sync_copy(zero_hbm.at[pl.ds(row0, ROWS_PER_TILE)],
                        z_sh.at[pl.ds(row0, ROWS_PER_TILE)])
        for i in range(NCHUNK):
            pltpu.make_async_copy(dst_hbm.at[pl.ds(base + i * CHUNK, CHUNK)],
                                  dst_v.at[i], psem).wait()
        pltpu.make_async_copy(src_hbm.at[pl.ds(toff, TAIL)], stail_v,
                              psem).wait()
        pltpu.make_async_copy(dst_hbm.at[pl.ds(toff, TAIL)], dtail_v,
                              psem).wait()
        idx_wait(0)
        gather_start(0)
        plsc.subcore_barrier()
        tail_g = pltpu.make_async_copy(y_hbm.at[stail_v], rtail_v, psem)
        tail_g.start()

        def process(ch, b, pf_gather, pf_idx, first, last):
            if pf_gather:
                idx_wait(1 - b)
                if not first:
                    scatter_wait(ch - 1, 1 - b)
                gather_start(1 - b)
            gather_wait(b)
            if pf_idx:
                idx_start(ch + 2, b)
            scatter_start(ch, b)
            if last:
                scatter_wait(ch, b)

        def body(g, _):
            process(g, 0, True, True, False, False)
            process(g + 1, 1, True, True, False, False)
            return ()

        process(0, 0, True, True, True, False)
        process(1, 1, True, True, False, False)
        lax.fori_loop(1, (NCHUNK - 2) // 2,
                      lambda i, _: body(i * 2, _), ())
        process(NCHUNK - 2, 0, True, False, False, False)
        process(NCHUNK - 1, 1, False, False, False, True)
        scatter_wait(NCHUNK - 2, 0)
        tail_g.wait()
        pltpu.sync_copy(rtail_v, z_sh.at[dtail_v], add=True)
        plsc.subcore_barrier()
        pltpu.sync_copy(z_sh.at[pl.ds(row0, ROWS_PER_TILE)],
                        out_hbm.at[c, pl.ds(row0, ROWS_PER_TILE)])

    return scatter_kernel


def _make_degree():
    mesh = plsc.VectorSubcoreMesh(core_axis_name="c", subcore_axis_name="s")

    @functools.partial(
        pl.kernel,
        out_type=jax.ShapeDtypeStruct((NW, N), jnp.float32),
        mesh=mesh,
        compiler_params=pltpu.CompilerParams(needs_layout_passes=False),
        scratch_types=[
            pltpu.VMEM((N,), jnp.float32),
            pltpu.VMEM((EPW,), jnp.int32),
        ],
    )
    def degree_kernel(dst_hbm, out_hbm, hist_v, dst_v):
        c = lax.axis_index("c")
        s = lax.axis_index("s")
        wid = s * NC + c
        pltpu.sync_copy(dst_hbm.at[pl.ds(wid * EPW, EPW)], dst_v)

        def zbody(i, _):
            hist_v[pl.ds(i * 16, 16)] = jnp.zeros((16,), jnp.float32)
            return ()

        lax.fori_loop(0, N // 16, zbody, ())
        ones16 = jnp.ones((16,), jnp.float32)

        def body(j, _):
            idx = dst_v[pl.ds(j * 16, 16)]
            plsc.addupdate_scatter(hist_v, [idx], ones16)
            return ()

        lax.fori_loop(0, EPW // 16, body, ())
        pltpu.sync_copy(hist_v, out_hbm.at[wid])

    return degree_kernel


_BR = 2048


def _tc_first(x, W1, deg_parts):
    def body(x_ref, w_ref, dp_ref, y_ref, dinv_ref):
        deg = jnp.sum(dp_ref[...], axis=0) + 1.0
        dinv = lax.rsqrt(deg)[:, None]
        xw = jnp.dot(x_ref[...], w_ref[...],
                     preferred_element_type=jnp.float32)
        y_ref[...] = dinv * xw
        dinv_ref[...] = dinv

    return pl.pallas_call(
        body,
        grid=(pl.cdiv(N, _BR),),
        in_specs=[
            pl.BlockSpec((_BR, D_IN), lambda i: (i, 0)),
            pl.BlockSpec((D_IN, D_H), lambda i: (0, 0)),
            pl.BlockSpec((NW, _BR), lambda i: (0, i)),
        ],
        out_specs=[
            pl.BlockSpec((_BR, D_H), lambda i: (i, 0)),
            pl.BlockSpec((_BR, 1), lambda i: (i, 0)),
        ],
        out_shape=[
            jax.ShapeDtypeStruct((N, D_H), jnp.float32),
            jax.ShapeDtypeStruct((N, 1), jnp.float32),
        ],
    )(x, W1, deg_parts)


def _tc_mid(z, y, dinv, b, W, d_out):
    d_in = y.shape[1]

    def body(z0_ref, z1_ref, y_ref, dinv_ref, b_ref, w_ref, out_ref):
        dv = dinv_ref[...]
        h = jnp.maximum(dv * (z0_ref[0] + z1_ref[0] + y_ref[...]) + b_ref[...],
                        0.0)
        out_ref[...] = dv * jnp.dot(h, w_ref[...],
                                    preferred_element_type=jnp.float32)

    return pl.pallas_call(
        body,
        grid=(pl.cdiv(N, _BR),),
        in_specs=[
            pl.BlockSpec((1, _BR, d_in), lambda i: (0, i, 0)),
            pl.BlockSpec((1, _BR, d_in), lambda i: (1, i, 0)),
            pl.BlockSpec((_BR, d_in), lambda i: (i, 0)),
            pl.BlockSpec((_BR, 1), lambda i: (i, 0)),
            pl.BlockSpec((1, d_in), lambda i: (0, 0)),
            pl.BlockSpec((d_in, d_out), lambda i: (0, 0)),
        ],
        out_specs=pl.BlockSpec((_BR, d_out), lambda i: (i, 0)),
        out_shape=jax.ShapeDtypeStruct((N, d_out), jnp.float32),
    )(z, z, y, dinv, b, W)


def _tc_last(z, y, dinv, b):
    def body(z0_ref, z1_ref, y_ref, dinv_ref, b_ref, out_ref):
        o = dinv_ref[...] * (z0_ref[0] + z1_ref[0] + y_ref[...]) + b_ref[...]
        m = jnp.max(o, axis=-1, keepdims=True)
        t = o - m
        out_ref[...] = t - jnp.log(jnp.sum(jnp.exp(t), axis=-1, keepdims=True))

    return pl.pallas_call(
        body,
        grid=(pl.cdiv(N, _BR),),
        in_specs=[
            pl.BlockSpec((1, _BR, D_OUT), lambda i: (0, i, 0)),
            pl.BlockSpec((1, _BR, D_OUT), lambda i: (1, i, 0)),
            pl.BlockSpec((_BR, D_OUT), lambda i: (i, 0)),
            pl.BlockSpec((_BR, 1), lambda i: (i, 0)),
            pl.BlockSpec((1, D_OUT), lambda i: (0, 0)),
        ],
        out_specs=pl.BlockSpec((_BR, D_OUT), lambda i: (i, 0)),
        out_shape=jax.ShapeDtypeStruct((N, D_OUT), jnp.float32),
    )(z, z, y, dinv, b)


def kernel(x, adj_t, W1, b1, W2, b2, W3, b3):
    src = adj_t[0]
    dst = adj_t[1]

    zeros128 = jnp.zeros((N_ACC, D_H), jnp.float32)
    zeros64 = jnp.zeros((N_ACC, D_OUT), jnp.float32)

    degree_k = _make_degree()
    scatter128 = _make_scatter(D_H)
    scatter64 = _make_scatter(D_OUT)

    deg_parts = degree_k(dst)

    y1, dinv = _tc_first(x, W1, deg_parts)
    z1 = scatter128(y1, src, dst, zeros128)
    y2 = _tc_mid(z1, y1, dinv, b1.reshape(1, D_H), W2, D_H)
    z2 = scatter128(y2, src, dst, zeros128)
    y3 = _tc_mid(z2, y2, dinv, b2.reshape(1, D_H), W3, D_OUT)
    z3 = scatter64(y3, src, dst, zeros64)

    return _tc_last(z3, y3, dinv, b3.reshape(1, D_OUT))


# speedup vs baseline: 32.4293x; 1.0028x over previous
import functools

import jax
import jax.numpy as jnp
from jax import lax
from jax.experimental import pallas as pl
from jax.experimental.pallas import tpu as pltpu
from jax.experimental.pallas import tpu_sc as plsc

N = 10000
E = 320000
D_IN = 128
D_H = 128
D_OUT = 64

NC = 2
NS = 16
NW = NC * NS
CHUNK = 128
TOTCH = E // CHUNK
NCHUNK = TOTCH // NW
XTRA = TOTCH - NCHUNK * NW
N_ACC = 10112
ROWS_PER_TILE = N_ACC // NS


def _make_scatter(D):
    mesh = plsc.VectorSubcoreMesh(core_axis_name="c", subcore_axis_name="s")
    params = None if D % 128 == 0 else pltpu.CompilerParams(
        use_tc_tiling_on_sc=False)

    @functools.partial(
        pl.kernel,
        out_type=jax.ShapeDtypeStruct((NC, N_ACC, D), jnp.float32),
        mesh=mesh,
        compiler_params=params,
        scratch_types=[
            [pltpu.VMEM((CHUNK,), jnp.int32)] * 2,
            pltpu.VMEM((NCHUNK, CHUNK), jnp.int32),
            [pltpu.VMEM((CHUNK, D), jnp.float32)] * 2,
            pltpu.VMEM((CHUNK,), jnp.int32),
            pltpu.VMEM((CHUNK,), jnp.int32),
            pltpu.VMEM_SHARED((N_ACC, D), jnp.float32),
            pltpu.SemaphoreType.DMA,
            [pltpu.SemaphoreType.DMA] * 2,
            [pltpu.SemaphoreType.DMA] * 2,
            [pltpu.SemaphoreType.DMA] * 2,
        ],
    )
    def scatter_kernel(y_hbm, e_hbm, zero_hbm, out_hbm,
                       src_v, dst_v, rows_v, sx_v, dx_v,
                       z_sh, psem, isem, gsem, ssem):
        c = lax.axis_index("c")
        s = lax.axis_index("s")
        wid = s * NC + c
        row0 = s * ROWS_PER_TILE
        cs = wid * NCHUNK + jnp.minimum(wid, XTRA)

        def idx_start(ch, b):
            pltpu.make_async_copy(
                e_hbm.at[pl.ds((cs + ch) * (2 * CHUNK), CHUNK)],
                src_v[b], isem[b]).start()

        def idx_wait(b):
            pltpu.make_async_copy(e_hbm.at[pl.ds(0, CHUNK)],
                                  src_v[b], isem[b]).wait()

        def gather_start(b):
            pltpu.make_async_copy(y_hbm.at[src_v[b]], rows_v[b],
                                  gsem[b]).start()

        def gather_wait(b):
            pltpu.make_async_copy(y_hbm.at[src_v[b]], rows_v[b],
                                  gsem[b]).wait()

        def scatter_start(ch, b):
            pltpu.make_async_copy(rows_v[b], z_sh.at[dst_v.at[ch]],
                                  ssem[b]).start(add=True)

        def scatter_wait(ch, b):
            pltpu.make_async_copy(rows_v[b], z_sh.at[dst_v.at[ch]],
                                  ssem[b]).wait()

        for i in range(NCHUNK):
            pltpu.make_async_copy(
                e_hbm.at[pl.ds((cs + i) * (2 * CHUNK) + CHUNK, CHUNK)],
                dst_v.at[i], psem).start()
        xoff = (cs + NCHUNK) * (2 * CHUNK)
        xgoff = jnp.where(wid < XTRA, xoff, 0)
        pltpu.make_async_copy(e_hbm.at[pl.ds(xgoff, CHUNK)], sx_v,
                              psem).start()
        pltpu.make_async_copy(e_hbm.at[pl.ds(xgoff + CHUNK, CHUNK)], dx_v,
                              psem).start()
        idx_start(0, 0)
        idx_start(1, 1)
        pltpu.sync_copy(zero_hbm.at[pl.ds(row0, ROWS_PER_TILE)],
                        z_sh.at[pl.ds(row0, ROWS_PER_TILE)])
        for i in range(NCHUNK):
            pltpu.make_async_copy(
                e_hbm.at[pl.ds((cs + i) * (2 * CHUNK) + CHUNK, CHUNK)],
                dst_v.at[i], psem).wait()
        pltpu.make_async_copy(e_hbm.at[pl.ds(xgoff, CHUNK)], sx_v,
                              psem).wait()
        pltpu.make_async_copy(e_hbm.at[pl.ds(xgoff + CHUNK, CHUNK)], dx_v,
                              psem).wait()
        idx_wait(0)
        gather_start(0)
        plsc.subcore_barrier()

        @pl.when(wid < XTRA)
        def _():
            pltpu.make_async_copy(y_hbm.at[sx_v], rows_v[1], psem).start()
            pltpu.make_async_copy(y_hbm.at[sx_v], rows_v[1], psem).wait()
            pltpu.sync_copy(rows_v[1], z_sh.at[dx_v], add=True)

        def process(ch, b, pf_gather, pf_idx, first, last):
            if pf_gather:
                idx_wait(1 - b)
                if not first:
                    scatter_wait(ch - 1, 1 - b)
                gather_start(1 - b)
            gather_wait(b)
            if pf_idx:
                idx_start(ch + 2, b)
            scatter_start(ch, b)
            if last:
                scatter_wait(ch, b)

        def body(g, _):
            process(g, 0, True, True, False, False)
            process(g + 1, 1, True, True, False, False)
            return ()

        process(0, 0, True, True, True, False)
        process(1, 1, True, True, False, False)
        lax.fori_loop(1, (NCHUNK - 2) // 2,
                      lambda i, _: body(i * 2, _), ())
        process(NCHUNK - 2, 0, True, False, False, False)
        process(NCHUNK - 1, 1, False, False, False, True)
        scatter_wait(NCHUNK - 2, 0)
        plsc.subcore_barrier()
        pltpu.sync_copy(z_sh.at[pl.ds(row0, ROWS_PER_TILE)],
                        out_hbm.at[c, pl.ds(row0, ROWS_PER_TILE)])

    return scatter_kernel


def _make_degree():
    mesh = plsc.VectorSubcoreMesh(core_axis_name="c", subcore_axis_name="s")

    @functools.partial(
        pl.kernel,
        out_type=jax.ShapeDtypeStruct((NW, N), jnp.float32),
        mesh=mesh,
        compiler_params=pltpu.CompilerParams(needs_layout_passes=False),
        scratch_types=[
            pltpu.VMEM((N,), jnp.float32),
            pltpu.VMEM((NCHUNK * CHUNK,), jnp.int32),
            pltpu.VMEM((CHUNK,), jnp.int32),
            pltpu.SemaphoreType.DMA,
        ],
    )
    def degree_kernel(e_hbm, out_hbm, hist_v, dst_v, dx_v, dsem):
        c = lax.axis_index("c")
        s = lax.axis_index("s")
        wid = s * NC + c
        cs = wid * NCHUNK + jnp.minimum(wid, XTRA)
        for i in range(NCHUNK):
            pltpu.make_async_copy(
                e_hbm.at[pl.ds((cs + i) * (2 * CHUNK) + CHUNK, CHUNK)],
                dst_v.at[pl.ds(i * CHUNK, CHUNK)], dsem).start()
        xoff = (cs + NCHUNK) * (2 * CHUNK)
        xgoff = jnp.where(wid < XTRA, xoff, 0)
        pltpu.make_async_copy(e_hbm.at[pl.ds(xgoff + CHUNK, CHUNK)], dx_v,
                              dsem).start()

        def zbody(i, _):
            hist_v[pl.ds(i * 16, 16)] = jnp.zeros((16,), jnp.float32)
            return ()

        lax.fori_loop(0, N // 16, zbody, ())
        for i in range(NCHUNK):
            pltpu.make_async_copy(
                e_hbm.at[pl.ds((cs + i) * (2 * CHUNK) + CHUNK, CHUNK)],
                dst_v.at[pl.ds(i * CHUNK, CHUNK)], dsem).wait()
        pltpu.make_async_copy(e_hbm.at[pl.ds(xgoff + CHUNK, CHUNK)], dx_v,
                              dsem).wait()
        ones16 = jnp.ones((16,), jnp.float32)

        def body(j, _):
            idx = dst_v[pl.ds(j * 16, 16)]
            plsc.addupdate_scatter(hist_v, [idx], ones16)
            return ()

        lax.fori_loop(0, (NCHUNK * CHUNK) // 16, body, ())

        @pl.when(wid < XTRA)
        def _():
            def xbody(j, _):
                idx = dx_v[pl.ds(j * 16, 16)]
                plsc.addupdate_scatter(hist_v, [idx], ones16)
                return ()
            lax.fori_loop(0, CHUNK // 16, xbody, ())

        pltpu.sync_copy(hist_v, out_hbm.at[wid])

    return degree_kernel


_BR = 2048


def _tc_first(x, W1, deg_parts):
    def body(x_ref, w_ref, dp_ref, y_ref, dinv_ref):
        deg = jnp.sum(dp_ref[...], axis=0) + 1.0
        dinv = lax.rsqrt(deg)[:, None]
        xw = jnp.dot(x_ref[...], w_ref[...],
                     preferred_element_type=jnp.float32)
        y_ref[...] = dinv * xw
        dinv_ref[...] = dinv

    return pl.pallas_call(
        body,
        grid=(pl.cdiv(N, _BR),),
        in_specs=[
            pl.BlockSpec((_BR, D_IN), lambda i: (i, 0)),
            pl.BlockSpec((D_IN, D_H), lambda i: (0, 0)),
            pl.BlockSpec((NW, _BR), lambda i: (0, i)),
        ],
        out_specs=[
            pl.BlockSpec((_BR, D_H), lambda i: (i, 0)),
            pl.BlockSpec((_BR, 1), lambda i: (i, 0)),
        ],
        out_shape=[
            jax.ShapeDtypeStruct((N, D_H), jnp.float32),
            jax.ShapeDtypeStruct((N, 1), jnp.float32),
        ],
    )(x, W1, deg_parts)


def _tc_mid(z, y, dinv, b, W, d_out):
    d_in = y.shape[1]

    def body(z0_ref, z1_ref, y_ref, dinv_ref, b_ref, w_ref, out_ref):
        dv = dinv_ref[...]
        h = jnp.maximum(dv * (z0_ref[0] + z1_ref[0] + y_ref[...]) + b_ref[...],
                        0.0)
        out_ref[...] = dv * jnp.dot(h, w_ref[...],
                                    preferred_element_type=jnp.float32)

    return pl.pallas_call(
        body,
        grid=(pl.cdiv(N, _BR),),
        in_specs=[
            pl.BlockSpec((1, _BR, d_in), lambda i: (0, i, 0)),
            pl.BlockSpec((1, _BR, d_in), lambda i: (1, i, 0)),
            pl.BlockSpec((_BR, d_in), lambda i: (i, 0)),
            pl.BlockSpec((_BR, 1), lambda i: (i, 0)),
            pl.BlockSpec((1, d_in), lambda i: (0, 0)),
            pl.BlockSpec((d_in, d_out), lambda i: (0, 0)),
        ],
        out_specs=pl.BlockSpec((_BR, d_out), lambda i: (i, 0)),
        out_shape=jax.ShapeDtypeStruct((N, d_out), jnp.float32),
    )(z, z, y, dinv, b, W)


def _tc_last(z, y, dinv, b):
    def body(z0_ref, z1_ref, y_ref, dinv_ref, b_ref, out_ref):
        o = dinv_ref[...] * (z0_ref[0] + z1_ref[0] + y_ref[...]) + b_ref[...]
        m = jnp.max(o, axis=-1, keepdims=True)
        t = o - m
        out_ref[...] = t - jnp.log(jnp.sum(jnp.exp(t), axis=-1, keepdims=True))

    return pl.pallas_call(
        body,
        grid=(pl.cdiv(N, _BR),),
        in_specs=[
            pl.BlockSpec((1, _BR, D_OUT), lambda i: (0, i, 0)),
            pl.BlockSpec((1, _BR, D_OUT), lambda i: (1, i, 0)),
            pl.BlockSpec((_BR, D_OUT), lambda i: (i, 0)),
            pl.BlockSpec((_BR, 1), lambda i: (i, 0)),
            pl.BlockSpec((1, D_OUT), lambda i: (0, 0)),
        ],
        out_specs=pl.BlockSpec((_BR, D_OUT), lambda i: (i, 0)),
        out_shape=jax.ShapeDtypeStruct((N, D_OUT), jnp.float32),
    )(z, z, y, dinv, b)


def kernel(x, adj_t, W1, b1, W2, b2, W3, b3):
    edges = adj_t.reshape(2, TOTCH, CHUNK).transpose(1, 0, 2).reshape(-1)

    zeros128 = jnp.zeros((N_ACC, D_H), jnp.float32)
    zeros64 = jnp.zeros((N_ACC, D_OUT), jnp.float32)

    degree_k = _make_degree()
    scatter128 = _make_scatter(D_H)
    scatter64 = _make_scatter(D_OUT)

    deg_parts = degree_k(edges)

    y1, dinv = _tc_first(x, W1, deg_parts)
    z1 = scatter128(y1, edges, zeros128)
    y2 = _tc_mid(z1, y1, dinv, b1.reshape(1, D_H), W2, D_H)
    z2 = scatter128(y2, edges, zeros128)
    y3 = _tc_mid(z2, y2, dinv, b2.reshape(1, D_H), W3, D_OUT)
    z3 = scatter64(y3, edges, zeros64)

    return _tc_last(z3, y3, dinv, b3.reshape(1, D_OUT))
